# Initial kernel scaffold; baseline (speedup 1.0000x reference)
import jax
import jax.numpy as jnp
from jax.experimental import pallas as pl


def kernel(x1, edge_index1, e1, u1, batch1, x2, edge_index2, e2, u2, batch2, params):
    raise NotImplementedError("write your pallas kernel here")


# speedup vs baseline: 1.3828x; 1.3828x over previous
import functools

import jax
import jax.numpy as jnp
from jax.experimental import pallas as pl

H = 32
F_E, F_X, F_U, F_OUT = 16, 128, 16, 8
N_NODES, N_EDGES, B = 10000, 320000, 16
N_ROUNDS, N_INNER = 2, 2


def _mlp(params, prefix, x):
    i = 0
    while f"{prefix}_W{i}" in params:
        x = x @ params[f"{prefix}_W{i}"] + params[f"{prefix}_b{i}"]
        if f"{prefix}_W{i + 1}" in params:
            x = jax.nn.relu(x)
        i += 1
    return x


def _attention_fast(aw, x, x_h, shared, batch, src, dest, gb_src, e_h_s, Ce, u, u_h):
    n = x.shape[0]
    sh_b = shared[batch]
    x_cat_var = jnp.concatenate([x_h, sh_b], axis=1)
    P = x @ aw["W0_x"] + x_cat_var @ aw["W0_xvar"]
    u_cat = jnp.concatenate([u, u_h, shared], axis=1)
    T16 = (shared @ aw["W0_esh"] + u_cat @ aw["W0_u"]
           - shared @ aw["W0_xsh"] + aw["b0"][None, :])
    xh_src = x_h[src]
    h0 = (P[dest] + Ce + e_h_s @ aw["W0_eh"]
          - xh_src @ aw["W0_xh"] + T16[gb_src])
    h = jax.nn.relu(h0)
    a = h[:, :64] @ aw["W1_a"] + aw["b1_a"]
    v = h[:, 64:] @ aw["W1_v"] + aw["b1_v"]
    ex = jnp.exp(a)
    y = ex * v
    s = jax.ops.segment_sum(ex, dest, num_segments=n)
    num = jax.ops.segment_sum(y, dest, num_segments=n)
    mx = jax.ops.segment_max(y, dest, num_segments=n)
    agg_sum = num / (s + 1e-16)
    agg_max = jnp.where(jnp.isneginf(mx), 0.0, mx / (s + 1e-16))
    node_in = jnp.concatenate([x, x_h, sh_b, agg_sum, agg_max, u_cat[batch]], axis=1)
    x_h_new = _mlp(aw["params"], "node", node_in)
    cnt = jnp.maximum(jax.ops.segment_sum(jnp.ones((n,), jnp.float32), batch, num_segments=B), 1.0)
    g_sum = jax.ops.segment_sum(x_h_new, batch, num_segments=B)
    g_max = jax.ops.segment_max(x_h_new, batch, num_segments=B)
    g_max = jnp.where(jnp.isneginf(g_max), 0.0, g_max)
    u_h_new = _mlp(aw["params"], "glob", jnp.concatenate([g_sum / cnt[:, None], g_max, u_cat], axis=1))
    return x_h_new, v, u_h_new


def kernel(x1, edge_index1, e1, u1, batch1, x2, edge_index2, e2, u2, batch2, params):
    fx = F_X + 2 * H
    W0 = jnp.concatenate([params["att_a_W0"], params["att_v_W0"]], axis=1)
    aw = {
        "params": params,
        "W0_x": W0[:F_X],
        "W0_xvar": W0[F_X:fx],
        "W0_xh": W0[F_X:F_X + H],
        "W0_xsh": W0[F_X + H:fx],
        "W0_e": W0[fx:fx + F_E],
        "W0_eh": W0[fx + F_E:fx + F_E + H],
        "W0_esh": W0[fx + F_E + H:fx + F_E + 2 * H],
        "W0_u": W0[fx + F_E + 2 * H:],
        "b0": jnp.concatenate([params["att_a_b0"], params["att_v_b0"]]),
        "W1_a": params["att_a_W1"], "b1_a": params["att_a_b1"],
        "W1_v": params["att_v_W1"], "b1_v": params["att_v_b1"],
    }

    def prep_graph(x, edge_index, e, batch):
        src, dest = edge_index[0], edge_index[1]
        perm = jnp.argsort(dest)
        src_s, dest_s = src[perm], dest[perm]
        e_s = e[perm]
        gb_src = batch[src_s]
        Ce = e_s @ aw["W0_e"] - (x @ aw["W0_x"])[src_s]
        return src_s, dest_s, e_s, gb_src, Ce

    g1 = prep_graph(x1, edge_index1, e1, batch1)
    g2 = prep_graph(x2, edge_index2, e2, batch2)

    x1h = _mlp(params, "enc_x", x1)
    x2h = _mlp(params, "enc_x", x2)
    e1h = _mlp(params, "enc_e", g1[2])
    e2h = _mlp(params, "enc_e", g2[2])
    u1h = _mlp(params, "enc_u", u1)
    u2h = _mlp(params, "enc_u", u2)

    outs = []
    for _ in range(N_ROUNDS):
        for _ in range(N_INNER):
            x1h, e1h, u1h = _attention_fast(aw, x1, x1h, u2h, batch1,
                                            g1[0], g1[1], g1[3], e1h, g1[4], u1, u1h)
        for _ in range(N_INNER):
            x2h, e2h, u2h = _attention_fast(aw, x2, x2h, u1h, batch2,
                                            g2[0], g2[1], g2[3], e2h, g2[4], u2, u2h)
        outs.append(_mlp(params, "dec", u2h))
    return jnp.stack(outs)


# speedup vs baseline: 2.3589x; 1.7059x over previous
import functools

import jax
import jax.numpy as jnp
from jax import lax
from jax.experimental import pallas as pl
from jax.experimental.pallas import tpu as pltpu

H = 32
F_E, F_X, F_U, F_OUT = 16, 128, 16, 8
N_NODES, N_EDGES, B = 10000, 320000, 16
N_ROUNDS, N_INNER = 2, 2


def _mlp(params, prefix, x):
    i = 0
    while f"{prefix}_W{i}" in params:
        x = x @ params[f"{prefix}_W{i}"] + params[f"{prefix}_b{i}"]
        if f"{prefix}_W{i + 1}" in params:
            x = jax.nn.relu(x)
        i += 1
    return x


_F32 = jnp.float32
_HI = jax.lax.Precision.HIGHEST


def _edge_block_body(K, R, d_lane_ref, d_col_ref, gb_ref, eh_ref, xh_ref, ce_ref,
                     P_ref, T16_ref, Weh_ref, Wxh_ref, W1a_ref, b1a_ref,
                     W1v_ref, b1v_ref, v_ref, accsum_ref, accmx_ref, pd_ref):
    b = pl.program_id(0)

    @pl.when(b == 0)
    def _init():
        accsum_ref[...] = jnp.zeros_like(accsum_ref)
        accmx_ref[...] = jnp.full_like(accmx_ref, -jnp.inf)

    dd = d_lane_ref[0]
    d0 = d_lane_ref[0, 0, 0]
    dlast = d_lane_ref[0, 0, K - 1]
    w0_init = (d0 // 8) * 8

    def _next_w0(w0):
        rem = jnp.where(dd >= w0 + R, dd, 2 ** 30)
        return (jnp.min(rem) // 8) * 8

    def _cond(w0):
        return w0 <= dlast

    iota_rk = lax.broadcasted_iota(jnp.int32, (R, K), 0)

    pd_ref[...] = jnp.zeros_like(pd_ref)

    def _gather_body(w0):
        M = (iota_rk == (dd - w0)).astype(_F32)
        Pwin = P_ref[pl.ds(w0, R), :]
        pd_ref[...] += lax.dot_general(M, Pwin, (((0,), (0,)), ((), ())),
                                       preferred_element_type=_F32,
                                       precision=_HI)
        return _next_w0(w0)

    lax.while_loop(_cond, _gather_body, w0_init)

    gb = gb_ref[0]
    oh16 = (lax.broadcasted_iota(jnp.int32, (16, K), 0) == gb).astype(_F32)
    tg = lax.dot_general(oh16, T16_ref[...], (((0,), (0,)), ((), ())),
                         preferred_element_type=_F32, precision=_HI)
    h0 = (ce_ref[...] + pd_ref[...] + tg
          + jnp.dot(eh_ref[...], Weh_ref[...], preferred_element_type=_F32,
                    precision=_HI)
          - jnp.dot(xh_ref[...], Wxh_ref[...], preferred_element_type=_F32,
                    precision=_HI))
    h = jnp.maximum(h0, 0.0)
    a = jnp.dot(h[:, :64], W1a_ref[...], preferred_element_type=_F32,
                precision=_HI) + b1a_ref[...]
    v = jnp.dot(h[:, 64:], W1v_ref[...], preferred_element_type=_F32,
                precision=_HI) + b1v_ref[...]
    v_ref[...] = v
    ex = jnp.exp(a)
    y = ex * v
    exy = jnp.concatenate([ex, y], axis=1)

    dcol = d_col_ref[0]
    m = y
    s = 1
    while s < K:
        dm = jnp.concatenate([jnp.full((s, H), -jnp.inf, _F32), m[:-s]], axis=0)
        dsh = jnp.concatenate([jnp.full((s, 1), -1, jnp.int32), dcol[:-s]], axis=0)
        m = jnp.where(dcol == dsh, jnp.maximum(m, dm), m)
        s *= 2
    dnext = jnp.concatenate([dd[:, 1:], jnp.full((1, 1), -5, jnp.int32)], axis=1)
    last = (dd != dnext).astype(_F32)

    def _scatter_body(w0):
        M = (iota_rk == (dd - w0)).astype(_F32)
        sn = jnp.dot(M, exy, preferred_element_type=_F32, precision=_HI)
        accsum_ref[pl.ds(w0, R), :] += sn
        M2 = M * last
        mxv = jnp.dot(M2, m, preferred_element_type=_F32, precision=_HI)
        has = jnp.sum(sn[:, :H], axis=1, keepdims=True) > 0.0
        upd = jnp.where(has, mxv, -jnp.inf)
        accmx_ref[pl.ds(w0, R), :] = jnp.maximum(accmx_ref[pl.ds(w0, R), :], upd)
        return _next_w0(w0)

    lax.while_loop(_cond, _scatter_body, w0_init)


@functools.partial(jax.jit, static_argnames=("E", "N_pad", "K", "R", "interpret"))
def _edge_call(d_lane, d_col, gb3, eh_s, xh_src, ce, P, T16, Weh, Wxh,
               W1a, b1a, W1v, b1v, *, E, N_pad, K, R, interpret=False):
    NB = E // K
    body = functools.partial(_edge_block_body, K, R)
    in_specs = [
            pl.BlockSpec((1, 1, K), lambda b: (b, 0, 0)),
            pl.BlockSpec((1, K, 1), lambda b: (b, 0, 0)),
            pl.BlockSpec((1, 1, K), lambda b: (b, 0, 0)),
            pl.BlockSpec((K, H), lambda b: (b, 0)),
            pl.BlockSpec((K, H), lambda b: (b, 0)),
            pl.BlockSpec((K, 128), lambda b: (b, 0)),
            pl.BlockSpec((N_pad, 128), lambda b: (0, 0)),
            pl.BlockSpec((16, 128), lambda b: (0, 0)),
            pl.BlockSpec((H, 128), lambda b: (0, 0)),
            pl.BlockSpec((H, 128), lambda b: (0, 0)),
            pl.BlockSpec((64, H), lambda b: (0, 0)),
            pl.BlockSpec((1, H), lambda b: (0, 0)),
            pl.BlockSpec((64, H), lambda b: (0, 0)),
            pl.BlockSpec((1, H), lambda b: (0, 0)),
    ]
    out_specs = [
        pl.BlockSpec((K, H), lambda b: (b, 0)),
        pl.BlockSpec((N_pad, 2 * H), lambda b: (0, 0)),
        pl.BlockSpec((N_pad, H), lambda b: (0, 0)),
    ]
    return pl.pallas_call(
        body,
        grid=(NB,),
        in_specs=in_specs,
        out_specs=out_specs,
        out_shape=[
            jax.ShapeDtypeStruct((E, H), _F32),
            jax.ShapeDtypeStruct((N_pad, 2 * H), _F32),
            jax.ShapeDtypeStruct((N_pad, H), _F32),
        ],
        scratch_shapes=[pltpu.VMEM((K, 128), _F32)],
        interpret=interpret,
    )(d_lane, d_col, gb3, eh_s, xh_src, ce, P, T16, Weh, Wxh, W1a, b1a, W1v, b1v)


_K_E = 512
_R_W = 128
_N_PAD = 10240
_USE_PALLAS_EDGE = True


def _attention_fast(aw, x, x_h, shared, batch, src, dest, gb_src, e_h_s, Ce, u, u_h,
                    blk):
    n = x.shape[0]
    sh_b = shared[batch]
    x_cat_var = jnp.concatenate([x_h, sh_b], axis=1)
    P = x @ aw["W0_x"] + x_cat_var @ aw["W0_xvar"]
    P_pad = jnp.pad(P, ((0, _N_PAD - n), (0, 0)))
    u_cat = jnp.concatenate([u, u_h, shared], axis=1)
    T16 = (shared @ aw["W0_esh"] + u_cat @ aw["W0_u"]
           - shared @ aw["W0_xsh"] + aw["b0"][None, :])
    xh_src = x_h[src]
    d_lane, d_col, gb3 = blk
    if not _USE_PALLAS_EDGE:
        h0 = (P[dest] + Ce + e_h_s @ aw["W0_eh"] - xh_src @ aw["W0_xh"]
              + T16[gb_src])
        h = jax.nn.relu(h0)
        a = h[:, :64] @ aw["W1_a"] + aw["b1_a"]
        v = h[:, 64:] @ aw["W1_v"] + aw["b1_v"]
        ex = jnp.exp(a)
        y = ex * v
        s = jax.ops.segment_sum(ex, dest, num_segments=n)
        num = jax.ops.segment_sum(y, dest, num_segments=n)
        mx = jax.ops.segment_max(y, dest, num_segments=n)
        agg_sum = num / (s + 1e-16)
        agg_max = jnp.where(jnp.isneginf(mx), 0.0, mx / (s + 1e-16))
        node_in = jnp.concatenate([x, x_h, sh_b, agg_sum, agg_max, u_cat[batch]], axis=1)
        x_h_new = _mlp(aw["params"], "node", node_in)
        cnt = jnp.maximum(jax.ops.segment_sum(jnp.ones((n,), jnp.float32), batch, num_segments=B), 1.0)
        g_sum = jax.ops.segment_sum(x_h_new, batch, num_segments=B)
        g_max = jax.ops.segment_max(x_h_new, batch, num_segments=B)
        g_max = jnp.where(jnp.isneginf(g_max), 0.0, g_max)
        u_h_new = _mlp(aw["params"], "glob", jnp.concatenate([g_sum / cnt[:, None], g_max, u_cat], axis=1))
        return x_h_new, v, u_h_new
    v, accsum, accmx = _edge_call(
        d_lane, d_col, gb3, e_h_s, xh_src, Ce, P_pad, T16,
        aw["W0_eh"], aw["W0_xh"], aw["W1_a"], aw["b1_a"], aw["W1_v"], aw["b1_v"],
        E=N_EDGES, N_pad=_N_PAD, K=_K_E, R=_R_W)
    s = accsum[:n, :H]
    num = accsum[:n, H:]
    mx = accmx[:n]
    agg_sum = num / (s + 1e-16)
    agg_max = jnp.where(jnp.isneginf(mx), 0.0, mx / (s + 1e-16))
    node_in = jnp.concatenate([x, x_h, sh_b, agg_sum, agg_max, u_cat[batch]], axis=1)
    x_h_new = _mlp(aw["params"], "node", node_in)
    cnt = jnp.maximum(jax.ops.segment_sum(jnp.ones((n,), jnp.float32), batch, num_segments=B), 1.0)
    g_sum = jax.ops.segment_sum(x_h_new, batch, num_segments=B)
    g_max = jax.ops.segment_max(x_h_new, batch, num_segments=B)
    g_max = jnp.where(jnp.isneginf(g_max), 0.0, g_max)
    u_h_new = _mlp(aw["params"], "glob", jnp.concatenate([g_sum / cnt[:, None], g_max, u_cat], axis=1))
    return x_h_new, v, u_h_new


def kernel(x1, edge_index1, e1, u1, batch1, x2, edge_index2, e2, u2, batch2, params):
    with jax.default_matmul_precision("highest"):
        return _kernel_impl(x1, edge_index1, e1, u1, batch1,
                            x2, edge_index2, e2, u2, batch2, params)


def _kernel_impl(x1, edge_index1, e1, u1, batch1, x2, edge_index2, e2, u2, batch2, params):
    fx = F_X + 2 * H
    W0 = jnp.concatenate([params["att_a_W0"], params["att_v_W0"]], axis=1)
    aw = {
        "params": params,
        "W0_x": W0[:F_X],
        "W0_xvar": W0[F_X:fx],
        "W0_xh": W0[F_X:F_X + H],
        "W0_xsh": W0[F_X + H:fx],
        "W0_e": W0[fx:fx + F_E],
        "W0_eh": W0[fx + F_E:fx + F_E + H],
        "W0_esh": W0[fx + F_E + H:fx + F_E + 2 * H],
        "W0_u": W0[fx + F_E + 2 * H:],
        "b0": jnp.concatenate([params["att_a_b0"], params["att_v_b0"]]),
        "W1_a": params["att_a_W1"], "b1_a": params["att_a_b1"].reshape(1, H),
        "W1_v": params["att_v_W1"], "b1_v": params["att_v_b1"].reshape(1, H),
    }

    def prep_graph(x, edge_index, e, batch):
        src, dest = edge_index[0], edge_index[1]
        perm = jnp.argsort(dest)
        src_s, dest_s = src[perm], dest[perm]
        e_s = e[perm]
        gb_src = batch[src_s]
        Ce = e_s @ aw["W0_e"] - (x @ aw["W0_x"])[src_s]
        nb = N_EDGES // _K_E
        blk = (dest_s.reshape(nb, 1, _K_E), dest_s.reshape(nb, _K_E, 1),
               gb_src.reshape(nb, 1, _K_E))
        return src_s, dest_s, e_s, gb_src, Ce, blk

    g1 = prep_graph(x1, edge_index1, e1, batch1)
    g2 = prep_graph(x2, edge_index2, e2, batch2)

    x1h = _mlp(params, "enc_x", x1)
    x2h = _mlp(params, "enc_x", x2)
    e1h = _mlp(params, "enc_e", g1[2])
    e2h = _mlp(params, "enc_e", g2[2])
    u1h = _mlp(params, "enc_u", u1)
    u2h = _mlp(params, "enc_u", u2)

    outs = []
    for _ in range(N_ROUNDS):
        for _ in range(N_INNER):
            x1h, e1h, u1h = _attention_fast(aw, x1, x1h, u2h, batch1,
                                            g1[0], g1[1], g1[3], e1h, g1[4], u1, u1h,
                                            g1[5])
        for _ in range(N_INNER):
            x2h, e2h, u2h = _attention_fast(aw, x2, x2h, u1h, batch2,
                                            g2[0], g2[1], g2[3], e2h, g2[4], u2, u2h,
                                            g2[5])
        outs.append(_mlp(params, "dec", u2h))
    return jnp.stack(outs)


# speedup vs baseline: 2.5923x; 1.0990x over previous
import functools

import jax
import jax.numpy as jnp
from jax import lax
from jax.experimental import pallas as pl
from jax.experimental.pallas import tpu as pltpu

H = 32
F_E, F_X, F_U, F_OUT = 16, 128, 16, 8
N_NODES, N_EDGES, B = 10000, 320000, 16
N_ROUNDS, N_INNER = 2, 2


def _mlp(params, prefix, x):
    i = 0
    while f"{prefix}_W{i}" in params:
        x = x @ params[f"{prefix}_W{i}"] + params[f"{prefix}_b{i}"]
        if f"{prefix}_W{i + 1}" in params:
            x = jax.nn.relu(x)
        i += 1
    return x


_F32 = jnp.float32
_HI = jax.lax.Precision.HIGHEST


def _edge_block_body(K, R, d_lane_ref, d_col_ref, gb_ref, eh_ref, xh_ref, ce_ref,
                     P_ref, T16_ref, Weh_ref, Wxh_ref, W1a_ref, b1a_ref,
                     W1v_ref, b1v_ref, v_ref, accsum_ref, accmx_ref, pd_ref):
    b = pl.program_id(0)

    @pl.when(b == 0)
    def _init():
        accsum_ref[...] = jnp.zeros_like(accsum_ref)
        accmx_ref[...] = jnp.full_like(accmx_ref, -jnp.inf)

    dd = d_lane_ref[0]
    d0 = d_lane_ref[0, 0, 0]
    dlast = d_lane_ref[0, 0, K - 1]
    w0_init = (d0 // 8) * 8

    def _next_w0(w0):
        rem = jnp.where(dd >= w0 + R, dd, 2 ** 30)
        return (jnp.min(rem) // 8) * 8

    def _cond(w0):
        return w0 <= dlast

    iota_rk = lax.broadcasted_iota(jnp.int32, (R, K), 0)

    pd_ref[...] = jnp.zeros_like(pd_ref)

    def _gather_body(w0):
        M = (iota_rk == (dd - w0)).astype(_F32)
        Pwin = P_ref[pl.ds(w0, R), :]
        pd_ref[...] += lax.dot_general(M, Pwin, (((0,), (0,)), ((), ())),
                                       preferred_element_type=_F32,
                                       precision=_HI)
        return _next_w0(w0)

    lax.while_loop(_cond, _gather_body, w0_init)

    gb = gb_ref[0]
    oh16 = (lax.broadcasted_iota(jnp.int32, (16, K), 0) == gb).astype(_F32)
    tg = lax.dot_general(oh16, T16_ref[...], (((0,), (0,)), ((), ())),
                         preferred_element_type=_F32, precision=_HI)
    h0 = (ce_ref[...] + pd_ref[...] + tg
          + jnp.dot(eh_ref[...], Weh_ref[...], preferred_element_type=_F32,
                    precision=_HI)
          - jnp.dot(xh_ref[...], Wxh_ref[...], preferred_element_type=_F32,
                    precision=_HI))
    h = jnp.maximum(h0, 0.0)
    a = jnp.dot(h[:, :64], W1a_ref[...], preferred_element_type=_F32,
                precision=_HI) + b1a_ref[...]
    v = jnp.dot(h[:, 64:], W1v_ref[...], preferred_element_type=_F32,
                precision=_HI) + b1v_ref[...]
    v_ref[...] = v
    ex = jnp.exp(a)
    y = ex * v
    exy = jnp.concatenate([ex, y], axis=1)

    dcol = d_col_ref[0]
    m = y
    s = 1
    while s < K:
        dm = jnp.concatenate([jnp.full((s, H), -jnp.inf, _F32), m[:-s]], axis=0)
        dsh = jnp.concatenate([jnp.full((s, 1), -1, jnp.int32), dcol[:-s]], axis=0)
        m = jnp.where(dcol == dsh, jnp.maximum(m, dm), m)
        s *= 2
    dnext = jnp.concatenate([dd[:, 1:], jnp.full((1, 1), -5, jnp.int32)], axis=1)
    last = (dd != dnext).astype(_F32)

    def _scatter_body(w0):
        M = (iota_rk == (dd - w0)).astype(_F32)
        sn = jnp.dot(M, exy, preferred_element_type=_F32, precision=_HI)
        accsum_ref[pl.ds(w0, R), :] += sn
        M2 = M * last
        mxv = jnp.dot(M2, m, preferred_element_type=_F32, precision=_HI)
        has = jnp.sum(sn[:, :H], axis=1, keepdims=True) > 0.0
        upd = jnp.where(has, mxv, -jnp.inf)
        accmx_ref[pl.ds(w0, R), :] = jnp.maximum(accmx_ref[pl.ds(w0, R), :], upd)
        return _next_w0(w0)

    lax.while_loop(_cond, _scatter_body, w0_init)


@functools.partial(jax.jit, static_argnames=("E", "N_pad", "K", "R", "interpret"))
def _edge_call(d_lane, d_col, gb3, eh_s, xh_src, ce, P, T16, Weh, Wxh,
               W1a, b1a, W1v, b1v, *, E, N_pad, K, R, interpret=False):
    NB = E // K
    body = functools.partial(_edge_block_body, K, R)
    in_specs = [
            pl.BlockSpec((1, 1, K), lambda b: (b, 0, 0)),
            pl.BlockSpec((1, K, 1), lambda b: (b, 0, 0)),
            pl.BlockSpec((1, 1, K), lambda b: (b, 0, 0)),
            pl.BlockSpec((K, H), lambda b: (b, 0)),
            pl.BlockSpec((K, H), lambda b: (b, 0)),
            pl.BlockSpec((K, 128), lambda b: (b, 0)),
            pl.BlockSpec((N_pad, 128), lambda b: (0, 0)),
            pl.BlockSpec((16, 128), lambda b: (0, 0)),
            pl.BlockSpec((H, 128), lambda b: (0, 0)),
            pl.BlockSpec((H, 128), lambda b: (0, 0)),
            pl.BlockSpec((64, H), lambda b: (0, 0)),
            pl.BlockSpec((1, H), lambda b: (0, 0)),
            pl.BlockSpec((64, H), lambda b: (0, 0)),
            pl.BlockSpec((1, H), lambda b: (0, 0)),
    ]
    out_specs = [
        pl.BlockSpec((K, H), lambda b: (b, 0)),
        pl.BlockSpec((N_pad, 2 * H), lambda b: (0, 0)),
        pl.BlockSpec((N_pad, H), lambda b: (0, 0)),
    ]
    return pl.pallas_call(
        body,
        grid=(NB,),
        in_specs=in_specs,
        out_specs=out_specs,
        out_shape=[
            jax.ShapeDtypeStruct((E, H), _F32),
            jax.ShapeDtypeStruct((N_pad, 2 * H), _F32),
            jax.ShapeDtypeStruct((N_pad, H), _F32),
        ],
        scratch_shapes=[pltpu.VMEM((K, 128), _F32)],
        interpret=interpret,
    )(d_lane, d_col, gb3, eh_s, xh_src, ce, P, T16, Weh, Wxh, W1a, b1a, W1v, b1v)


_K_E = 2560
_R_W = 128
_N_PAD = 10240
_USE_PALLAS_EDGE = True


def _attention_fast(aw, x, x_h, shared, batch, src, dest, gb_src, e_h_s, Ce, u, u_h,
                    blk):
    n = x.shape[0]
    sh_b = shared[batch]
    x_cat_var = jnp.concatenate([x_h, sh_b], axis=1)
    P = x @ aw["W0_x"] + x_cat_var @ aw["W0_xvar"]
    P_pad = jnp.pad(P, ((0, _N_PAD - n), (0, 0)))
    u_cat = jnp.concatenate([u, u_h, shared], axis=1)
    T16 = (shared @ aw["W0_esh"] + u_cat @ aw["W0_u"]
           - shared @ aw["W0_xsh"] + aw["b0"][None, :])
    xh_src = x_h[src]
    d_lane, d_col, gb3 = blk
    if not _USE_PALLAS_EDGE:
        h0 = (P[dest] + Ce + e_h_s @ aw["W0_eh"] - xh_src @ aw["W0_xh"]
              + T16[gb_src])
        h = jax.nn.relu(h0)
        a = h[:, :64] @ aw["W1_a"] + aw["b1_a"]
        v = h[:, 64:] @ aw["W1_v"] + aw["b1_v"]
        ex = jnp.exp(a)
        y = ex * v
        s = jax.ops.segment_sum(ex, dest, num_segments=n)
        num = jax.ops.segment_sum(y, dest, num_segments=n)
        mx = jax.ops.segment_max(y, dest, num_segments=n)
        agg_sum = num / (s + 1e-16)
        agg_max = jnp.where(jnp.isneginf(mx), 0.0, mx / (s + 1e-16))
        node_in = jnp.concatenate([x, x_h, sh_b, agg_sum, agg_max, u_cat[batch]], axis=1)
        x_h_new = _mlp(aw["params"], "node", node_in)
        cnt = jnp.maximum(jax.ops.segment_sum(jnp.ones((n,), jnp.float32), batch, num_segments=B), 1.0)
        g_sum = jax.ops.segment_sum(x_h_new, batch, num_segments=B)
        g_max = jax.ops.segment_max(x_h_new, batch, num_segments=B)
        g_max = jnp.where(jnp.isneginf(g_max), 0.0, g_max)
        u_h_new = _mlp(aw["params"], "glob", jnp.concatenate([g_sum / cnt[:, None], g_max, u_cat], axis=1))
        return x_h_new, v, u_h_new
    v, accsum, accmx = _edge_call(
        d_lane, d_col, gb3, e_h_s, xh_src, Ce, P_pad, T16,
        aw["W0_eh"], aw["W0_xh"], aw["W1_a"], aw["b1_a"], aw["W1_v"], aw["b1_v"],
        E=N_EDGES, N_pad=_N_PAD, K=_K_E, R=_R_W)
    s = accsum[:n, :H]
    num = accsum[:n, H:]
    mx = accmx[:n]
    agg_sum = num / (s + 1e-16)
    agg_max = jnp.where(jnp.isneginf(mx), 0.0, mx / (s + 1e-16))
    node_in = jnp.concatenate([x, x_h, sh_b, agg_sum, agg_max, u_cat[batch]], axis=1)
    x_h_new = _mlp(aw["params"], "node", node_in)
    cnt = jnp.maximum(jax.ops.segment_sum(jnp.ones((n,), jnp.float32), batch, num_segments=B), 1.0)
    g_sum = jax.ops.segment_sum(x_h_new, batch, num_segments=B)
    g_max = jax.ops.segment_max(x_h_new, batch, num_segments=B)
    g_max = jnp.where(jnp.isneginf(g_max), 0.0, g_max)
    u_h_new = _mlp(aw["params"], "glob", jnp.concatenate([g_sum / cnt[:, None], g_max, u_cat], axis=1))
    return x_h_new, v, u_h_new


def kernel(x1, edge_index1, e1, u1, batch1, x2, edge_index2, e2, u2, batch2, params):
    with jax.default_matmul_precision("highest"):
        return _kernel_impl(x1, edge_index1, e1, u1, batch1,
                            x2, edge_index2, e2, u2, batch2, params)


def _kernel_impl(x1, edge_index1, e1, u1, batch1, x2, edge_index2, e2, u2, batch2, params):
    fx = F_X + 2 * H
    W0 = jnp.concatenate([params["att_a_W0"], params["att_v_W0"]], axis=1)
    aw = {
        "params": params,
        "W0_x": W0[:F_X],
        "W0_xvar": W0[F_X:fx],
        "W0_xh": W0[F_X:F_X + H],
        "W0_xsh": W0[F_X + H:fx],
        "W0_e": W0[fx:fx + F_E],
        "W0_eh": W0[fx + F_E:fx + F_E + H],
        "W0_esh": W0[fx + F_E + H:fx + F_E + 2 * H],
        "W0_u": W0[fx + F_E + 2 * H:],
        "b0": jnp.concatenate([params["att_a_b0"], params["att_v_b0"]]),
        "W1_a": params["att_a_W1"], "b1_a": params["att_a_b1"].reshape(1, H),
        "W1_v": params["att_v_W1"], "b1_v": params["att_v_b1"].reshape(1, H),
    }

    def prep_graph(x, edge_index, e, batch):
        src, dest = edge_index[0], edge_index[1]
        perm = jnp.argsort(dest)
        src_s, dest_s = src[perm], dest[perm]
        e_s = e[perm]
        gb_src = batch[src_s]
        Ce = e_s @ aw["W0_e"] - (x @ aw["W0_x"])[src_s]
        nb = N_EDGES // _K_E
        blk = (dest_s.reshape(nb, 1, _K_E), dest_s.reshape(nb, _K_E, 1),
               gb_src.reshape(nb, 1, _K_E))
        return src_s, dest_s, e_s, gb_src, Ce, blk

    g1 = prep_graph(x1, edge_index1, e1, batch1)
    g2 = prep_graph(x2, edge_index2, e2, batch2)

    x1h = _mlp(params, "enc_x", x1)
    x2h = _mlp(params, "enc_x", x2)
    e1h = _mlp(params, "enc_e", g1[2])
    e2h = _mlp(params, "enc_e", g2[2])
    u1h = _mlp(params, "enc_u", u1)
    u2h = _mlp(params, "enc_u", u2)

    outs = []
    for _ in range(N_ROUNDS):
        for _ in range(N_INNER):
            x1h, e1h, u1h = _attention_fast(aw, x1, x1h, u2h, batch1,
                                            g1[0], g1[1], g1[3], e1h, g1[4], u1, u1h,
                                            g1[5])
        for _ in range(N_INNER):
            x2h, e2h, u2h = _attention_fast(aw, x2, x2h, u1h, batch2,
                                            g2[0], g2[1], g2[3], e2h, g2[4], u2, u2h,
                                            g2[5])
        outs.append(_mlp(params, "dec", u2h))
    return jnp.stack(outs)


# speedup vs baseline: 3.1876x; 1.2296x over previous
import functools

import jax
import jax.numpy as jnp
from jax import lax
from jax.experimental import pallas as pl
from jax.experimental.pallas import tpu as pltpu

H = 32
F_E, F_X, F_U, F_OUT = 16, 128, 16, 8
N_NODES, N_EDGES, B = 10000, 320000, 16
N_ROUNDS, N_INNER = 2, 2


def _mlp(params, prefix, x):
    i = 0
    while f"{prefix}_W{i}" in params:
        x = x @ params[f"{prefix}_W{i}"] + params[f"{prefix}_b{i}"]
        if f"{prefix}_W{i + 1}" in params:
            x = jax.nn.relu(x)
        i += 1
    return x


_F32 = jnp.float32
_HI = jax.lax.Precision.HIGHEST


def _edge_block_body(K, R, d_lane_ref, d_col_ref, gb_ref, eh_ref, xh_ref, ce_ref,
                     P_ref, T16_ref, Weh_ref, Wxh_ref, W1a_ref, b1a_ref,
                     W1v_ref, b1v_ref, v_ref, accsum_ref, accmx_ref, pd_ref):
    b = pl.program_id(0)

    @pl.when(b == 0)
    def _init():
        accsum_ref[...] = jnp.zeros_like(accsum_ref)
        accmx_ref[...] = jnp.full_like(accmx_ref, -jnp.inf)

    dd = d_lane_ref[0]
    d0 = d_lane_ref[0, 0, 0]
    dlast = d_lane_ref[0, 0, K - 1]
    w0_init = (d0 // 8) * 8

    def _next_w0(w0):
        rem = jnp.where(dd >= w0 + R, dd, 2 ** 30)
        return (jnp.min(rem) // 8) * 8

    def _cond(w0):
        return w0 <= dlast

    iota_rk = lax.broadcasted_iota(jnp.int32, (R, K), 0)

    pd_ref[...] = jnp.zeros_like(pd_ref)

    def _gather_body(w0):
        M = (iota_rk == (dd - w0)).astype(_F32)
        Pwin = P_ref[pl.ds(w0, R), :]
        pd_ref[...] += lax.dot_general(M, Pwin, (((0,), (0,)), ((), ())),
                                       preferred_element_type=_F32,
                                       precision=_HI)
        return _next_w0(w0)

    lax.while_loop(_cond, _gather_body, w0_init)

    gb = gb_ref[0]
    oh16 = (lax.broadcasted_iota(jnp.int32, (16, K), 0) == gb).astype(_F32)
    tg = lax.dot_general(oh16, T16_ref[...], (((0,), (0,)), ((), ())),
                         preferred_element_type=_F32, precision=_HI)
    h0 = (ce_ref[...] + pd_ref[...] + tg
          + jnp.dot(eh_ref[...], Weh_ref[...], preferred_element_type=_F32,
                    precision=_HI)
          - jnp.dot(xh_ref[...], Wxh_ref[...], preferred_element_type=_F32,
                    precision=_HI))
    h = jnp.maximum(h0, 0.0)
    a = jnp.dot(h[:, :64], W1a_ref[...], preferred_element_type=_F32,
                precision=_HI) + b1a_ref[...]
    v = jnp.dot(h[:, 64:], W1v_ref[...], preferred_element_type=_F32,
                precision=_HI) + b1v_ref[...]
    v_ref[...] = v
    ex = jnp.exp(a)
    y = ex * v
    exy = jnp.concatenate([ex, y], axis=1)

    dcol = d_col_ref[0]
    m = y
    s = 1
    while s < K:
        dm = jnp.concatenate([jnp.full((s, H), -jnp.inf, _F32), m[:-s]], axis=0)
        dsh = jnp.concatenate([jnp.full((s, 1), -1, jnp.int32), dcol[:-s]], axis=0)
        m = jnp.where(dcol == dsh, jnp.maximum(m, dm), m)
        s *= 2
    dnext = jnp.concatenate([dd[:, 1:], jnp.full((1, 1), -5, jnp.int32)], axis=1)
    last = (dd != dnext).astype(_F32)

    def _scatter_body(w0):
        M = (iota_rk == (dd - w0)).astype(_F32)
        sn = jnp.dot(M, exy, preferred_element_type=_F32, precision=_HI)
        accsum_ref[pl.ds(w0, R), :] += sn
        M2 = M * last
        mxv = jnp.dot(M2, m, preferred_element_type=_F32, precision=_HI)
        has = jnp.sum(sn[:, :H], axis=1, keepdims=True) > 0.0
        upd = jnp.where(has, mxv, -jnp.inf)
        accmx_ref[pl.ds(w0, R), :] = jnp.maximum(accmx_ref[pl.ds(w0, R), :], upd)
        return _next_w0(w0)

    lax.while_loop(_cond, _scatter_body, w0_init)


@functools.partial(jax.jit, static_argnames=("E", "N_pad", "K", "R", "interpret"))
def _edge_call(d_lane, d_col, gb3, eh_s, xh_src, ce, P, T16, Weh, Wxh,
               W1a, b1a, W1v, b1v, *, E, N_pad, K, R, interpret=False):
    NB = E // K
    body = functools.partial(_edge_block_body, K, R)
    in_specs = [
            pl.BlockSpec((1, 1, K), lambda b: (b, 0, 0)),
            pl.BlockSpec((1, K, 1), lambda b: (b, 0, 0)),
            pl.BlockSpec((1, 1, K), lambda b: (b, 0, 0)),
            pl.BlockSpec((K, H), lambda b: (b, 0)),
            pl.BlockSpec((K, H), lambda b: (b, 0)),
            pl.BlockSpec((K, 128), lambda b: (b, 0)),
            pl.BlockSpec((N_pad, 128), lambda b: (0, 0)),
            pl.BlockSpec((16, 128), lambda b: (0, 0)),
            pl.BlockSpec((H, 128), lambda b: (0, 0)),
            pl.BlockSpec((H, 128), lambda b: (0, 0)),
            pl.BlockSpec((64, H), lambda b: (0, 0)),
            pl.BlockSpec((1, H), lambda b: (0, 0)),
            pl.BlockSpec((64, H), lambda b: (0, 0)),
            pl.BlockSpec((1, H), lambda b: (0, 0)),
    ]
    out_specs = [
        pl.BlockSpec((K, H), lambda b: (b, 0)),
        pl.BlockSpec((N_pad, 2 * H), lambda b: (0, 0)),
        pl.BlockSpec((N_pad, H), lambda b: (0, 0)),
    ]
    return pl.pallas_call(
        body,
        grid=(NB,),
        in_specs=in_specs,
        out_specs=out_specs,
        out_shape=[
            jax.ShapeDtypeStruct((E, H), _F32),
            jax.ShapeDtypeStruct((N_pad, 2 * H), _F32),
            jax.ShapeDtypeStruct((N_pad, H), _F32),
        ],
        scratch_shapes=[pltpu.VMEM((K, 128), _F32)],
        interpret=interpret,
    )(d_lane, d_col, gb3, eh_s, xh_src, ce, P, T16, Weh, Wxh, W1a, b1a, W1v, b1v)


try:
    from jax.experimental.pallas import tpu_sc as plsc
    _HAVE_SC = True
except ImportError:
    _HAVE_SC = False


@functools.partial(jax.jit, static_argnames=("D", "C"))
def _sc_gather(table, idx, *, D, C):
    E = idx.shape[0]
    NC, NS = 2, 16
    NW = NC * NS
    per_w = E // NW
    S = per_w // C
    mesh = plsc.VectorSubcoreMesh(core_axis_name="c", subcore_axis_name="s")

    @functools.partial(
        pl.kernel, mesh=mesh,
        out_type=jax.ShapeDtypeStruct((E, D), jnp.float32),
        compiler_params=pltpu.CompilerParams(use_tc_tiling_on_sc=False),
        scratch_types=[
            pltpu.VMEM((C,), jnp.int32),
            pltpu.VMEM((C, D), jnp.float32),
            pltpu.SemaphoreType.DMA,
        ],
    )
    def k(table_hbm, idx_hbm, out_hbm, idx_v, rows_v, sem):
        wid = lax.axis_index("s") * NC + lax.axis_index("c")
        for j in range(S):
            base = wid * per_w + j * C
            pltpu.sync_copy(idx_hbm.at[pl.ds(base, C)], idx_v)
            pltpu.async_copy(table_hbm.at[idx_v], rows_v, sem).wait()
            pltpu.sync_copy(rows_v, out_hbm.at[pl.ds(base, C)])

    return k(table, idx)


_K_E = 2560
_R_W = 128
_N_PAD = 10240
_USE_PALLAS_EDGE = True


def _attention_fast(aw, x, x_h, shared, batch, src, dest, gb_src, e_h_s, Ce, u, u_h,
                    blk):
    n = x.shape[0]
    sh_b = shared[batch]
    x_cat_var = jnp.concatenate([x_h, sh_b], axis=1)
    P = x @ aw["W0_x"] + x_cat_var @ aw["W0_xvar"]
    P_pad = jnp.pad(P, ((0, _N_PAD - n), (0, 0)))
    u_cat = jnp.concatenate([u, u_h, shared], axis=1)
    T16 = (shared @ aw["W0_esh"] + u_cat @ aw["W0_u"]
           - shared @ aw["W0_xsh"] + aw["b0"][None, :])
    xh_src = _sc_gather(x_h, src, D=32, C=2000) if _HAVE_SC else x_h[src]
    d_lane, d_col, gb3 = blk
    if not _USE_PALLAS_EDGE:
        h0 = (P[dest] + Ce + e_h_s @ aw["W0_eh"] - xh_src @ aw["W0_xh"]
              + T16[gb_src])
        h = jax.nn.relu(h0)
        a = h[:, :64] @ aw["W1_a"] + aw["b1_a"]
        v = h[:, 64:] @ aw["W1_v"] + aw["b1_v"]
        ex = jnp.exp(a)
        y = ex * v
        s = jax.ops.segment_sum(ex, dest, num_segments=n)
        num = jax.ops.segment_sum(y, dest, num_segments=n)
        mx = jax.ops.segment_max(y, dest, num_segments=n)
        agg_sum = num / (s + 1e-16)
        agg_max = jnp.where(jnp.isneginf(mx), 0.0, mx / (s + 1e-16))
        node_in = jnp.concatenate([x, x_h, sh_b, agg_sum, agg_max, u_cat[batch]], axis=1)
        x_h_new = _mlp(aw["params"], "node", node_in)
        cnt = jnp.maximum(jax.ops.segment_sum(jnp.ones((n,), jnp.float32), batch, num_segments=B), 1.0)
        g_sum = jax.ops.segment_sum(x_h_new, batch, num_segments=B)
        g_max = jax.ops.segment_max(x_h_new, batch, num_segments=B)
        g_max = jnp.where(jnp.isneginf(g_max), 0.0, g_max)
        u_h_new = _mlp(aw["params"], "glob", jnp.concatenate([g_sum / cnt[:, None], g_max, u_cat], axis=1))
        return x_h_new, v, u_h_new
    v, accsum, accmx = _edge_call(
        d_lane, d_col, gb3, e_h_s, xh_src, Ce, P_pad, T16,
        aw["W0_eh"], aw["W0_xh"], aw["W1_a"], aw["b1_a"], aw["W1_v"], aw["b1_v"],
        E=N_EDGES, N_pad=_N_PAD, K=_K_E, R=_R_W)
    s = accsum[:n, :H]
    num = accsum[:n, H:]
    mx = accmx[:n]
    agg_sum = num / (s + 1e-16)
    agg_max = jnp.where(jnp.isneginf(mx), 0.0, mx / (s + 1e-16))
    node_in = jnp.concatenate([x, x_h, sh_b, agg_sum, agg_max, u_cat[batch]], axis=1)
    x_h_new = _mlp(aw["params"], "node", node_in)
    cnt = jnp.maximum(jax.ops.segment_sum(jnp.ones((n,), jnp.float32), batch, num_segments=B), 1.0)
    g_sum = jax.ops.segment_sum(x_h_new, batch, num_segments=B)
    g_max = jax.ops.segment_max(x_h_new, batch, num_segments=B)
    g_max = jnp.where(jnp.isneginf(g_max), 0.0, g_max)
    u_h_new = _mlp(aw["params"], "glob", jnp.concatenate([g_sum / cnt[:, None], g_max, u_cat], axis=1))
    return x_h_new, v, u_h_new


def kernel(x1, edge_index1, e1, u1, batch1, x2, edge_index2, e2, u2, batch2, params):
    with jax.default_matmul_precision("highest"):
        return _kernel_impl(x1, edge_index1, e1, u1, batch1,
                            x2, edge_index2, e2, u2, batch2, params)


def _kernel_impl(x1, edge_index1, e1, u1, batch1, x2, edge_index2, e2, u2, batch2, params):
    fx = F_X + 2 * H
    W0 = jnp.concatenate([params["att_a_W0"], params["att_v_W0"]], axis=1)
    aw = {
        "params": params,
        "W0_x": W0[:F_X],
        "W0_xvar": W0[F_X:fx],
        "W0_xh": W0[F_X:F_X + H],
        "W0_xsh": W0[F_X + H:fx],
        "W0_e": W0[fx:fx + F_E],
        "W0_eh": W0[fx + F_E:fx + F_E + H],
        "W0_esh": W0[fx + F_E + H:fx + F_E + 2 * H],
        "W0_u": W0[fx + F_E + 2 * H:],
        "b0": jnp.concatenate([params["att_a_b0"], params["att_v_b0"]]),
        "W1_a": params["att_a_W1"], "b1_a": params["att_a_b1"].reshape(1, H),
        "W1_v": params["att_v_W1"], "b1_v": params["att_v_b1"].reshape(1, H),
    }

    def prep_graph(x, edge_index, e, batch):
        src, dest = edge_index[0], edge_index[1]
        perm = jnp.argsort(dest)
        src_s, dest_s = src[perm], dest[perm]
        e_s = _sc_gather(e, perm, D=16, C=2000) if _HAVE_SC else e[perm]
        gb_src = batch[src_s]
        Px = x @ aw["W0_x"]
        Px_s = _sc_gather(Px, src_s, D=128, C=1000) if _HAVE_SC else Px[src_s]
        Ce = e_s @ aw["W0_e"] - Px_s
        nb = N_EDGES // _K_E
        blk = (dest_s.reshape(nb, 1, _K_E), dest_s.reshape(nb, _K_E, 1),
               gb_src.reshape(nb, 1, _K_E))
        return src_s, dest_s, e_s, gb_src, Ce, blk

    g1 = prep_graph(x1, edge_index1, e1, batch1)
    g2 = prep_graph(x2, edge_index2, e2, batch2)

    x1h = _mlp(params, "enc_x", x1)
    x2h = _mlp(params, "enc_x", x2)
    e1h = _mlp(params, "enc_e", g1[2])
    e2h = _mlp(params, "enc_e", g2[2])
    u1h = _mlp(params, "enc_u", u1)
    u2h = _mlp(params, "enc_u", u2)

    outs = []
    for _ in range(N_ROUNDS):
        for _ in range(N_INNER):
            x1h, e1h, u1h = _attention_fast(aw, x1, x1h, u2h, batch1,
                                            g1[0], g1[1], g1[3], e1h, g1[4], u1, u1h,
                                            g1[5])
        for _ in range(N_INNER):
            x2h, e2h, u2h = _attention_fast(aw, x2, x2h, u1h, batch2,
                                            g2[0], g2[1], g2[3], e2h, g2[4], u2, u2h,
                                            g2[5])
        outs.append(_mlp(params, "dec", u2h))
    return jnp.stack(outs)


# speedup vs baseline: 3.1884x; 1.0003x over previous
import functools

import jax
import jax.numpy as jnp
from jax import lax
from jax.experimental import pallas as pl
from jax.experimental.pallas import tpu as pltpu

H = 32
F_E, F_X, F_U, F_OUT = 16, 128, 16, 8
N_NODES, N_EDGES, B = 10000, 320000, 16
N_ROUNDS, N_INNER = 2, 2


def _mlp(params, prefix, x):
    i = 0
    while f"{prefix}_W{i}" in params:
        x = x @ params[f"{prefix}_W{i}"] + params[f"{prefix}_b{i}"]
        if f"{prefix}_W{i + 1}" in params:
            x = jax.nn.relu(x)
        i += 1
    return x


_F32 = jnp.float32
_HI = jax.lax.Precision.HIGHEST


def _edge_block_body(K, R, d_lane_ref, d_col_ref, gb_ref, eh_ref, xh_ref, ce_ref,
                     P_ref, T16_ref, Weh_ref, Wxh_ref, W1a_ref, b1a_ref,
                     W1v_ref, b1v_ref, v_ref, accsum_ref, accmx_ref, pd_ref):
    b = pl.program_id(0)

    @pl.when(b == 0)
    def _init():
        accsum_ref[...] = jnp.zeros_like(accsum_ref)
        accmx_ref[...] = jnp.full_like(accmx_ref, -jnp.inf)

    dd = d_lane_ref[0]
    d0 = d_lane_ref[0, 0, 0]
    dlast = d_lane_ref[0, 0, K - 1]
    w0_init = (d0 // 8) * 8

    def _next_w0(w0):
        rem = jnp.where(dd >= w0 + R, dd, 2 ** 30)
        return (jnp.min(rem) // 8) * 8

    def _cond(w0):
        return w0 <= dlast

    iota_rk = lax.broadcasted_iota(jnp.int32, (R, K), 0)

    pd_ref[...] = jnp.zeros_like(pd_ref)

    def _gather_body(w0):
        M = (iota_rk == (dd - w0)).astype(_F32)
        Pwin = P_ref[pl.ds(w0, R), :]
        pd_ref[...] += lax.dot_general(M, Pwin, (((0,), (0,)), ((), ())),
                                       preferred_element_type=_F32,
                                       precision=_HI)
        return _next_w0(w0)

    lax.while_loop(_cond, _gather_body, w0_init)

    gb = gb_ref[0]
    oh16 = (lax.broadcasted_iota(jnp.int32, (16, K), 0) == gb).astype(_F32)
    tg = lax.dot_general(oh16, T16_ref[...], (((0,), (0,)), ((), ())),
                         preferred_element_type=_F32, precision=_HI)
    h0 = (ce_ref[...] + pd_ref[...] + tg
          + jnp.dot(eh_ref[...], Weh_ref[...], preferred_element_type=_F32,
                    precision=_HI)
          - jnp.dot(xh_ref[...], Wxh_ref[...], preferred_element_type=_F32,
                    precision=_HI))
    h = jnp.maximum(h0, 0.0)
    a = jnp.dot(h[:, :64], W1a_ref[...], preferred_element_type=_F32,
                precision=_HI) + b1a_ref[...]
    v = jnp.dot(h[:, 64:], W1v_ref[...], preferred_element_type=_F32,
                precision=_HI) + b1v_ref[...]
    v_ref[...] = v
    ex = jnp.exp(a)
    y = ex * v
    exy = jnp.concatenate([ex, y], axis=1)

    dcol = d_col_ref[0]
    m = y
    s = 1
    while s < K:
        dm = jnp.concatenate([jnp.full((s, H), -jnp.inf, _F32), m[:-s]], axis=0)
        dsh = jnp.concatenate([jnp.full((s, 1), -1, jnp.int32), dcol[:-s]], axis=0)
        m = jnp.where(dcol == dsh, jnp.maximum(m, dm), m)
        s *= 2
    dnext = jnp.concatenate([dd[:, 1:], jnp.full((1, 1), -5, jnp.int32)], axis=1)
    last = (dd != dnext).astype(_F32)

    def _scatter_body(w0):
        M = (iota_rk == (dd - w0)).astype(_F32)
        sn = jnp.dot(M, exy, preferred_element_type=_F32, precision=_HI)
        accsum_ref[pl.ds(w0, R), :] += sn
        M2 = M * last
        mxv = jnp.dot(M2, m, preferred_element_type=_F32, precision=_HI)
        has = jnp.sum(sn[:, :H], axis=1, keepdims=True) > 0.0
        upd = jnp.where(has, mxv, -jnp.inf)
        accmx_ref[pl.ds(w0, R), :] = jnp.maximum(accmx_ref[pl.ds(w0, R), :], upd)
        return _next_w0(w0)

    lax.while_loop(_cond, _scatter_body, w0_init)


@functools.partial(jax.jit, static_argnames=("E", "N_pad", "K", "R", "interpret"))
def _edge_call(d_lane, d_col, gb3, eh_s, xh_src, ce, P, T16, Weh, Wxh,
               W1a, b1a, W1v, b1v, *, E, N_pad, K, R, interpret=False):
    NB = E // K
    body = functools.partial(_edge_block_body, K, R)
    in_specs = [
            pl.BlockSpec((1, 1, K), lambda b: (b, 0, 0)),
            pl.BlockSpec((1, K, 1), lambda b: (b, 0, 0)),
            pl.BlockSpec((1, 1, K), lambda b: (b, 0, 0)),
            pl.BlockSpec((K, H), lambda b: (b, 0)),
            pl.BlockSpec((K, H), lambda b: (b, 0)),
            pl.BlockSpec((K, 128), lambda b: (b, 0)),
            pl.BlockSpec((N_pad, 128), lambda b: (0, 0)),
            pl.BlockSpec((16, 128), lambda b: (0, 0)),
            pl.BlockSpec((H, 128), lambda b: (0, 0)),
            pl.BlockSpec((H, 128), lambda b: (0, 0)),
            pl.BlockSpec((64, H), lambda b: (0, 0)),
            pl.BlockSpec((1, H), lambda b: (0, 0)),
            pl.BlockSpec((64, H), lambda b: (0, 0)),
            pl.BlockSpec((1, H), lambda b: (0, 0)),
    ]
    out_specs = [
        pl.BlockSpec((K, H), lambda b: (b, 0)),
        pl.BlockSpec((N_pad, 2 * H), lambda b: (0, 0)),
        pl.BlockSpec((N_pad, H), lambda b: (0, 0)),
    ]
    return pl.pallas_call(
        body,
        grid=(NB,),
        in_specs=in_specs,
        out_specs=out_specs,
        out_shape=[
            jax.ShapeDtypeStruct((E, H), _F32),
            jax.ShapeDtypeStruct((N_pad, 2 * H), _F32),
            jax.ShapeDtypeStruct((N_pad, H), _F32),
        ],
        scratch_shapes=[pltpu.VMEM((K, 128), _F32)],
        interpret=interpret,
    )(d_lane, d_col, gb3, eh_s, xh_src, ce, P, T16, Weh, Wxh, W1a, b1a, W1v, b1v)


from jax.experimental.pallas import tpu_sc as plsc


@functools.partial(jax.jit, static_argnames=("D", "C"))
def _sc_gather(table, idx, *, D, C):
    E = idx.shape[0]
    NC, NS = 2, 16
    NW = NC * NS
    per_w = E // NW
    S = per_w // C
    mesh = plsc.VectorSubcoreMesh(core_axis_name="c", subcore_axis_name="s")

    @functools.partial(
        pl.kernel, mesh=mesh,
        out_type=jax.ShapeDtypeStruct((E, D), jnp.float32),
        compiler_params=pltpu.CompilerParams(use_tc_tiling_on_sc=False),
        scratch_types=[
            pltpu.VMEM((C,), jnp.int32),
            pltpu.VMEM((C, D), jnp.float32),
            pltpu.SemaphoreType.DMA,
        ],
    )
    def k(table_hbm, idx_hbm, out_hbm, idx_v, rows_v, sem):
        wid = lax.axis_index("s") * NC + lax.axis_index("c")
        for j in range(S):
            base = wid * per_w + j * C
            pltpu.sync_copy(idx_hbm.at[pl.ds(base, C)], idx_v)
            pltpu.async_copy(table_hbm.at[idx_v], rows_v, sem).wait()
            pltpu.sync_copy(rows_v, out_hbm.at[pl.ds(base, C)])

    return k(table, idx)


_K_E = 2560
_R_W = 128
_N_PAD = 10240


def _attention_fast(aw, x, x_h, shared, batch, src, dest, gb_src, e_h_s, Ce, u, u_h,
                    blk):
    n = x.shape[0]
    sh_b = shared[batch]
    x_cat_var = jnp.concatenate([x_h, sh_b], axis=1)
    P = x @ aw["W0_x"] + x_cat_var @ aw["W0_xvar"]
    P_pad = jnp.pad(P, ((0, _N_PAD - n), (0, 0)))
    u_cat = jnp.concatenate([u, u_h, shared], axis=1)
    T16 = (shared @ aw["W0_esh"] + u_cat @ aw["W0_u"]
           - shared @ aw["W0_xsh"] + aw["b0"][None, :])
    xh_src = _sc_gather(x_h, src, D=32, C=2000)
    d_lane, d_col, gb3 = blk
    v, accsum, accmx = _edge_call(
        d_lane, d_col, gb3, e_h_s, xh_src, Ce, P_pad, T16,
        aw["W0_eh"], aw["W0_xh"], aw["W1_a"], aw["b1_a"], aw["W1_v"], aw["b1_v"],
        E=N_EDGES, N_pad=_N_PAD, K=_K_E, R=_R_W)
    s = accsum[:n, :H]
    num = accsum[:n, H:]
    mx = accmx[:n]
    agg_sum = num / (s + 1e-16)
    agg_max = jnp.where(jnp.isneginf(mx), 0.0, mx / (s + 1e-16))
    node_in = jnp.concatenate([x, x_h, sh_b, agg_sum, agg_max, u_cat[batch]], axis=1)
    x_h_new = _mlp(aw["params"], "node", node_in)
    cnt = jnp.maximum(jax.ops.segment_sum(jnp.ones((n,), jnp.float32), batch, num_segments=B), 1.0)
    g_sum = jax.ops.segment_sum(x_h_new, batch, num_segments=B)
    g_max = jax.ops.segment_max(x_h_new, batch, num_segments=B)
    g_max = jnp.where(jnp.isneginf(g_max), 0.0, g_max)
    u_h_new = _mlp(aw["params"], "glob", jnp.concatenate([g_sum / cnt[:, None], g_max, u_cat], axis=1))
    return x_h_new, v, u_h_new


def kernel(x1, edge_index1, e1, u1, batch1, x2, edge_index2, e2, u2, batch2, params):
    with jax.default_matmul_precision("highest"):
        return _kernel_impl(x1, edge_index1, e1, u1, batch1,
                            x2, edge_index2, e2, u2, batch2, params)


def _kernel_impl(x1, edge_index1, e1, u1, batch1, x2, edge_index2, e2, u2, batch2, params):
    fx = F_X + 2 * H
    W0 = jnp.concatenate([params["att_a_W0"], params["att_v_W0"]], axis=1)
    aw = {
        "params": params,
        "W0_x": W0[:F_X],
        "W0_xvar": W0[F_X:fx],
        "W0_xh": W0[F_X:F_X + H],
        "W0_xsh": W0[F_X + H:fx],
        "W0_e": W0[fx:fx + F_E],
        "W0_eh": W0[fx + F_E:fx + F_E + H],
        "W0_esh": W0[fx + F_E + H:fx + F_E + 2 * H],
        "W0_u": W0[fx + F_E + 2 * H:],
        "b0": jnp.concatenate([params["att_a_b0"], params["att_v_b0"]]),
        "W1_a": params["att_a_W1"], "b1_a": params["att_a_b1"].reshape(1, H),
        "W1_v": params["att_v_W1"], "b1_v": params["att_v_b1"].reshape(1, H),
    }

    def prep_graph(x, edge_index, e, batch):
        src, dest = edge_index[0], edge_index[1]
        perm = jnp.argsort(dest)
        src_s, dest_s = src[perm], dest[perm]
        e_s = _sc_gather(e, perm, D=16, C=2000)
        gb_src = batch[src_s]
        Px = x @ aw["W0_x"]
        Px_s = _sc_gather(Px, src_s, D=128, C=1000)
        Ce = e_s @ aw["W0_e"] - Px_s
        nb = N_EDGES // _K_E
        blk = (dest_s.reshape(nb, 1, _K_E), dest_s.reshape(nb, _K_E, 1),
               gb_src.reshape(nb, 1, _K_E))
        return src_s, dest_s, e_s, gb_src, Ce, blk

    g1 = prep_graph(x1, edge_index1, e1, batch1)
    g2 = prep_graph(x2, edge_index2, e2, batch2)

    x1h = _mlp(params, "enc_x", x1)
    x2h = _mlp(params, "enc_x", x2)
    e1h = _mlp(params, "enc_e", g1[2])
    e2h = _mlp(params, "enc_e", g2[2])
    u1h = _mlp(params, "enc_u", u1)
    u2h = _mlp(params, "enc_u", u2)

    outs = []
    for _ in range(N_ROUNDS):
        for _ in range(N_INNER):
            x1h, e1h, u1h = _attention_fast(aw, x1, x1h, u2h, batch1,
                                            g1[0], g1[1], g1[3], e1h, g1[4], u1, u1h,
                                            g1[5])
        for _ in range(N_INNER):
            x2h, e2h, u2h = _attention_fast(aw, x2, x2h, u1h, batch2,
                                            g2[0], g2[1], g2[3], e2h, g2[4], u2, u2h,
                                            g2[5])
        outs.append(_mlp(params, "dec", u2h))
    return jnp.stack(outs)
